# Initial kernel scaffold; baseline (speedup 1.0000x reference)
import jax
import jax.numpy as jnp
from jax.experimental import pallas as pl


def kernel(asset_scores, short_ratio):
    raise NotImplementedError("write your pallas kernel here")


# speedup vs baseline: 9.8373x; 9.8373x over previous
import jax
import jax.numpy as jnp
from jax.experimental import pallas as pl

G = 1024
ROWS_PER_BLOCK = 8


def _select_softmax_body(x_ref, long_ref, short_ref):
    s = x_ref[...]
    z = jnp.where(s == 0.0, 0.0, s)
    b = jax.lax.bitcast_convert_type(z, jnp.int32)
    key = jnp.where(b < 0, b ^ jnp.int32(0x7FFFFFFF), b)
    keyn = ~key

    def cnt_ge(k, t):
        return jnp.sum((k >= t).astype(jnp.int32), axis=1, keepdims=True)

    imin = jnp.int32(-2147483648)
    zero = jnp.int32(0)
    tw0 = jnp.where(cnt_ge(key, zero) >= G, zero, imin)
    tl0 = jnp.where(cnt_ge(keyn, zero) >= G, zero, imin)

    def step(i, carry):
        tw, tl = carry
        bit = jnp.int32(1) << (jnp.int32(30) - i)
        cw = tw | bit
        cl = tl | bit
        tw = jnp.where(cnt_ge(key, cw) >= G, cw, tw)
        tl = jnp.where(cnt_ge(keyn, cl) >= G, cl, tl)
        return tw, tl

    tw, tl = jax.lax.fori_loop(0, 31, step, (tw0, tl0))

    R, N = s.shape
    iota = jax.lax.broadcasted_iota(jnp.int32, (R, N), 1)
    riota = jnp.int32(N - 1) - iota

    gt_w = key > tw
    eq_w = key == tw
    eq_wi = eq_w.astype(jnp.int32)
    need_w = G - jnp.sum(gt_w.astype(jnp.int32), axis=1, keepdims=True)

    gt_l = keyn > tl
    eq_l = keyn == tl
    eq_li = eq_l.astype(jnp.int32)
    need_l = G - jnp.sum(gt_l.astype(jnp.int32), axis=1, keepdims=True)

    def idx_step(i, carry):
        jw, jl = carry
        bit = jnp.int32(1) << (jnp.int32(14) - i)
        cw = jw | bit
        cl = jl | bit
        fw = jnp.sum(jnp.where(iota <= cw, eq_wi, 0), axis=1, keepdims=True)
        fl = jnp.sum(jnp.where(riota <= cl, eq_li, 0), axis=1, keepdims=True)
        jw = jnp.where(fw <= need_w, cw, jw)
        jl = jnp.where(fl <= need_l, cl, jl)
        return jw, jl

    j0 = jnp.zeros((R, 1), jnp.int32)
    jw, jl = jax.lax.fori_loop(0, 15, idx_step, (j0, j0))

    sel_w = gt_w | (eq_w & (iota <= jw))
    sel_l = gt_l | (eq_l & (riota <= jl))

    m = jnp.max(s, axis=1, keepdims=True)
    mn = jnp.min(s, axis=1, keepdims=True)
    ew = jnp.where(sel_w, jnp.exp(s - m), 0.0)
    el = jnp.where(sel_l, jnp.exp(mn - s), 0.0)
    sw = jnp.sum(ew, axis=1, keepdims=True)
    sl = jnp.sum(el, axis=1, keepdims=True)
    long_ref[...] = ew / sw
    short_ref[...] = el / sl


def kernel(asset_scores, short_ratio):
    B, N = asset_scores.shape
    R = ROWS_PER_BLOCK
    spec = pl.BlockSpec((R, N), lambda i: (i, 0))
    long_w, short_w = pl.pallas_call(
        _select_softmax_body,
        grid=(B // R,),
        in_specs=[spec],
        out_specs=[spec, spec],
        out_shape=[
            jax.ShapeDtypeStruct((B, N), asset_scores.dtype),
            jax.ShapeDtypeStruct((B, N), asset_scores.dtype),
        ],
    )(asset_scores)
    return (long_w, short_w, jnp.clip(short_ratio, 0.0, 1.0))


# speedup vs baseline: 15.4139x; 1.5669x over previous
import jax
import jax.numpy as jnp
from jax.experimental import pallas as pl

G = 1024
ROWS_PER_BLOCK = 8


def _select_softmax_body(x_ref, long_ref, short_ref):
    s = x_ref[...]
    R, N = s.shape
    z = jnp.where(s == 0.0, 0.0, s)
    b = jax.lax.bitcast_convert_type(z, jnp.int32)
    key = jnp.where(b < 0, b ^ jnp.int32(0x7FFFFFFF), b)
    keyn = ~key

    def cnt_ge(k, t):
        return jnp.sum((k >= t).astype(jnp.int32), axis=1, keepdims=True)

    imin = jnp.int32(-2147483648)
    zero = jnp.int32(0)
    gi = jnp.int32(G)

    c0w = cnt_ge(key, zero)
    c0l = cnt_ge(keyn, zero)
    tw0 = jnp.where(c0w >= gi, zero, imin)
    tl0 = jnp.where(c0l >= gi, zero, imin)
    dw0 = (c0w == gi).astype(jnp.int32)
    dl0 = (c0l == gi).astype(jnp.int32)
    sw0 = jnp.where(dw0 == 1, zero, imin)
    sl0 = jnp.where(dl0 == 1, zero, imin)
    nrows = jnp.int32(2 * c0w.shape[0])

    def cond(carry):
        i, tw, tl, dw, sw, dl, sl = carry
        return (i < 31) & (jnp.sum(dw) + jnp.sum(dl) < nrows)

    def body(carry):
        i, tw, tl, dw, sw, dl, sl = carry
        bit = jnp.int32(1) << (jnp.int32(30) - i)
        cw = tw | bit
        cl = tl | bit
        ccw = cnt_ge(key, cw)
        ccl = cnt_ge(keyn, cl)
        tw = jnp.where(ccw >= gi, cw, tw)
        tl = jnp.where(ccl >= gi, cl, tl)
        hitw = (ccw == gi) & (dw == 0)
        hitl = (ccl == gi) & (dl == 0)
        sw = jnp.where(hitw, cw, sw)
        sl = jnp.where(hitl, cl, sl)
        return i + 1, tw, tl, dw | hitw.astype(jnp.int32), sw, dl | hitl.astype(jnp.int32), sl

    i0 = jnp.int32(0)
    _, tw, tl, dw, sw, dl, sl = jax.lax.while_loop(
        cond, body, (i0, tw0, tl0, dw0, sw0, dl0, sl0))

    iota = jax.lax.broadcasted_iota(jnp.int32, (R, N), 1)
    riota = jnp.int32(N - 1) - iota

    gt_w = key > tw
    eq_w = key == tw
    eq_wi = eq_w.astype(jnp.int32)
    need_w = gi - jnp.sum(gt_w.astype(jnp.int32), axis=1, keepdims=True)

    gt_l = keyn > tl
    eq_l = keyn == tl
    eq_li = eq_l.astype(jnp.int32)
    need_l = gi - jnp.sum(gt_l.astype(jnp.int32), axis=1, keepdims=True)

    def idx_cond(carry):
        i, jw, jl = carry
        return (i < 15) & (jnp.sum(dw) + jnp.sum(dl) < nrows)

    def idx_body(carry):
        i, jw, jl = carry
        bit = jnp.int32(1) << (jnp.int32(14) - i)
        cw = jw | bit
        cl = jl | bit
        fw = jnp.sum(jnp.where(iota <= cw, eq_wi, 0), axis=1, keepdims=True)
        fl = jnp.sum(jnp.where(riota <= cl, eq_li, 0), axis=1, keepdims=True)
        jw = jnp.where(fw <= need_w, cw, jw)
        jl = jnp.where(fl <= need_l, cl, jl)
        return i + 1, jw, jl

    j0 = jnp.zeros((R, 1), jnp.int32)
    _, jw, jl = jax.lax.while_loop(idx_cond, idx_body, (i0, j0, j0))

    sel_wi = jnp.where(dw == 1, (key >= sw).astype(jnp.int32),
                       (gt_w | (eq_w & (iota <= jw))).astype(jnp.int32))
    sel_li = jnp.where(dl == 1, (keyn >= sl).astype(jnp.int32),
                       (gt_l | (eq_l & (riota <= jl))).astype(jnp.int32))
    sel_w = sel_wi == 1
    sel_l = sel_li == 1

    m = jnp.max(s, axis=1, keepdims=True)
    mn = jnp.min(s, axis=1, keepdims=True)
    ew = jnp.where(sel_w, jnp.exp(s - m), 0.0)
    el = jnp.where(sel_l, jnp.exp(mn - s), 0.0)
    sum_w = jnp.sum(ew, axis=1, keepdims=True)
    sum_l = jnp.sum(el, axis=1, keepdims=True)
    long_ref[...] = ew / sum_w
    short_ref[...] = el / sum_l


def kernel(asset_scores, short_ratio):
    B, N = asset_scores.shape
    R = ROWS_PER_BLOCK
    spec = pl.BlockSpec((R, N), lambda i: (i, 0))
    long_w, short_w = pl.pallas_call(
        _select_softmax_body,
        grid=(B // R,),
        in_specs=[spec],
        out_specs=[spec, spec],
        out_shape=[
            jax.ShapeDtypeStruct((B, N), asset_scores.dtype),
            jax.ShapeDtypeStruct((B, N), asset_scores.dtype),
        ],
    )(asset_scores)
    return (long_w, short_w, jnp.clip(short_ratio, 0.0, 1.0))
